# baseline (speedup 1.0000x reference)
import functools

import jax
import jax.numpy as jnp
from jax import lax
from jax.experimental import pallas as pl
from jax.experimental.pallas import tpu as pltpu
from jax.experimental.pallas import tpu_sc as plsc

_V = 1000000
_D = 64
_B = 4096
_T = 200
_SR = _V // 2
_AC = 384
_NA = _V // _AC
_AREM = _NA * _AC
_BC = 256
_TG = 8
_NB = (_T // _TG) * (_B // _BC)


def _mesh():
    return plsc.VectorSubcoreMesh(core_axis_name="c", subcore_axis_name="s")


@functools.cache
def _build():
    info = plsc.get_sparse_core_info()
    nc = info.num_cores
    nw = nc * info.num_subcores

    @functools.partial(
        pl.kernel,
        mesh=_mesh(),
        out_type=jax.ShapeDtypeStruct((_SR, 128), jnp.float32),
        scratch_types=[
            pltpu.VMEM((_D, _AC), jnp.float32),
            pltpu.VMEM((_D, _AC), jnp.float32),
            pltpu.VMEM((_AC // 2, 128), jnp.float32),
            pltpu.VMEM((_AC // 2, 128), jnp.float32),
            pltpu.VMEM((32, 128), jnp.float32),
            pltpu.SemaphoreType.DMA,
            pltpu.SemaphoreType.DMA,
            pltpu.SemaphoreType.DMA,
            pltpu.SemaphoreType.DMA,
        ],
        compiler_params=pltpu.CompilerParams(
            use_tc_tiling_on_sc=True, needs_layout_passes=False
        ),
    )
    def a_kernel(tt_hbm, tail_hbm, scr_hbm, stage0_v, stage1_v, rowsT0_v,
                 rowsT1_v, tail_v, a0sem, a1sem, o0sem, o1sem):
        wid = lax.axis_index("s") * nc + lax.axis_index("c")
        lio = lax.iota(jnp.int32, 16)
        perms = [(lio + k) & 15 for k in range(16)]
        n_mine = (_NA - wid + nw - 1) // nw
        n_pairs = n_mine // 2

        def in_copy(m, stage, sem):
            i0 = pl.multiple_of((wid + m * nw) * _AC, _AC)
            return pltpu.make_async_copy(tt_hbm.at[:, pl.ds(i0, _AC)], stage,
                                         sem)

        def out_copy(m, rowsT, sem):
            r0 = pl.multiple_of((wid + m * nw) * (_AC // 2), _AC // 2)
            return pltpu.make_async_copy(rowsT, scr_hbm.at[pl.ds(r0, _AC // 2), :],
                                         sem)

        def transpose(stage, rowsT):
            @plsc.parallel_loop(0, (_AC // 16) * 4, unroll=4)
            def block(q):
                jb = q // 4
                db = q % 4
                d_vec = db * 16 + lio
                jb16 = jnp.broadcast_to(jb * 16, (16,)).astype(jnp.int32)
                jb8 = jnp.broadcast_to(jb * 8, (16,)).astype(jnp.int32)
                for k in range(16):
                    j_vec = jb16 + perms[k]
                    v = plsc.load_gather(stage, [d_vec, j_vec])
                    row = jb8 + lax.shift_right_logical(perms[k], 1)
                    col = lax.shift_left(perms[k] & 1, 6) + d_vec
                    plsc.store_scatter(rowsT, [row, col], v)

        in_copy(0, stage0_v, a0sem).start()

        def pair(g, carry):
            m0 = 2 * g
            m1 = 2 * g + 1
            in_copy(m0, stage0_v, a0sem).wait()
            in_copy(m1, stage1_v, a1sem).start()

            @pl.when(g > 0)
            def _():
                out_copy(m0, rowsT0_v, o0sem).wait()

            transpose(stage0_v, rowsT0_v)
            out_copy(m0, rowsT0_v, o0sem).start()

            in_copy(m1, stage1_v, a1sem).wait()

            @pl.when(m1 + 1 < n_mine)
            def _():
                in_copy(m1 + 1, stage0_v, a0sem).start()

            @pl.when(g > 0)
            def _():
                out_copy(m1, rowsT1_v, o1sem).wait()

            transpose(stage1_v, rowsT1_v)
            out_copy(m1, rowsT1_v, o1sem).start()
            return carry

        lax.fori_loop(0, n_pairs, pair, 0)

        @pl.when(n_mine % 2 == 1)
        def _():
            mt = 2 * n_pairs
            in_copy(mt, stage0_v, a0sem).wait()

            @pl.when(n_pairs > 0)
            def _():
                out_copy(mt, rowsT0_v, o0sem).wait()

            transpose(stage0_v, rowsT0_v)
            out_copy(mt, rowsT0_v, o0sem).start()

        out_copy(0, rowsT0_v, o0sem).wait()

        @pl.when(n_pairs > 0)
        def _():
            out_copy(0, rowsT1_v, o1sem).wait()

        @pl.when(wid == nw - 1)
        def _():
            pltpu.sync_copy(tail_hbm, tail_v)
            pltpu.sync_copy(tail_v, scr_hbm.at[pl.ds(_AREM // 2, 32), :])

    @functools.partial(
        pl.kernel,
        mesh=_mesh(),
        out_type=jax.ShapeDtypeStruct((_T, _D, _B), jnp.float32),
        scratch_types=[
            pltpu.VMEM((_TG, _BC), jnp.int32),
            pltpu.VMEM((_TG, _BC), jnp.int32),
            pltpu.VMEM((_TG, _BC), jnp.int32),
            pltpu.VMEM((_BC, 128), jnp.float32),
            pltpu.VMEM((_BC, 128), jnp.float32),
            pltpu.VMEM((_D, _BC), jnp.float32),
            pltpu.VMEM((_D, _BC), jnp.float32),
            pltpu.SemaphoreType.DMA,
            pltpu.SemaphoreType.DMA,
            pltpu.SemaphoreType.DMA,
            pltpu.SemaphoreType.DMA,
        ],
        compiler_params=pltpu.CompilerParams(
            use_tc_tiling_on_sc=True, needs_layout_passes=False
        ),
    )
    def b_kernel(scr_hbm, ids_hbm, out_hbm, ids_v, idx2_v, par_v, rows0_v,
                 rows1_v, piece0_v, piece1_v, g0sem, g1sem, p0sem, p1sem):
        wid = lax.axis_index("s") * nc + lax.axis_index("c")
        lio = lax.iota(jnp.int32, 16)
        perms = [(lio + k) & 15 for k in range(16)]
        n_mine = (_NB - wid + nw - 1) // nw

        def unit(m, _):
            u = wid + m * nw
            g = u // (_B // _BC)
            bc = u % (_B // _BC)
            t0 = pl.multiple_of(g * _TG, _TG)
            b0 = pl.multiple_of(bc * _BC, _BC)
            pltpu.sync_copy(ids_hbm.at[pl.ds(t0, _TG), pl.ds(b0, _BC)], ids_v)

            @plsc.parallel_loop(0, _TG * (_BC // 16), unroll=4)
            def prep(q):
                r = q // (_BC // 16)
                c = q % (_BC // 16)
                v = ids_v[r, pl.ds(16 * c, 16)]
                idx2_v[r, pl.ds(16 * c, 16)] = lax.shift_right_logical(v, 1)
                par_v[r, pl.ds(16 * c, 16)] = lax.bitwise_and(v, 1)

            def g_fire(tl, rows, sem):
                for k in range(_BC // 128):
                    pltpu.async_copy(
                        scr_hbm.at[idx2_v.at[tl, pl.ds(128 * k, 128)]],
                        rows.at[pl.ds(128 * k, 128)],
                        sem,
                    )

            def g_wait(tl, rows, sem):
                for k in range(_BC // 128):
                    pltpu.make_async_copy(
                        scr_hbm.at[idx2_v.at[tl, pl.ds(128 * k, 128)]],
                        rows.at[pl.ds(128 * k, 128)],
                        sem,
                    ).wait()

            def out_slice(tl):
                return out_hbm.at[t0 + tl, :, pl.ds(b0, _BC)]

            g_fire(0, rows0_v, g0sem)
            for tl in range(_TG):
                rows, gs = (rows0_v, g0sem) if tl % 2 == 0 else (rows1_v, g1sem)
                piece, ps = ((piece0_v, p0sem) if tl % 2 == 0
                             else (piece1_v, p1sem))
                g_wait(tl, rows, gs)
                if tl + 1 < _TG:
                    nrows, ngs = ((rows1_v, g1sem) if tl % 2 == 0
                                  else (rows0_v, g0sem))
                    g_fire(tl + 1, nrows, ngs)
                if tl >= 2:
                    pltpu.make_async_copy(piece, out_slice(tl - 2), ps).wait()

                @plsc.parallel_loop(0, (_D // 16) * (_BC // 16), unroll=8)
                def bb_body(q):
                    db = q // (_BC // 16)
                    bb = q % (_BC // 16)
                    row = 16 * bb + lio
                    par64 = lax.shift_left(par_v[tl, pl.ds(16 * bb, 16)], 6)
                    for k in range(16):
                        dperm = db * 16 + perms[k]
                        v = plsc.load_gather(rows, [row, par64 + dperm])
                        plsc.store_scatter(piece, [dperm, row], v)

                pltpu.async_copy(piece, out_slice(tl), ps)
            pltpu.make_async_copy(piece0_v, out_slice(_TG - 2), p0sem).wait()
            pltpu.make_async_copy(piece1_v, out_slice(_TG - 1), p1sem).wait()
            return _

        lax.fori_loop(0, n_mine, unit, 0)

    return a_kernel, b_kernel


def kernel(input_ids, table):
    a_kernel, b_kernel = _build()
    tt = table.T
    tail = table[_AREM:].reshape(32, 128)
    scratch = a_kernel(tt, tail)
    out_t = b_kernel(scratch, input_ids.T)
    return jnp.transpose(out_t, (2, 0, 1))


# speedup vs baseline: 1.0083x; 1.0083x over previous
import functools

import jax
import jax.numpy as jnp
from jax import lax
from jax.experimental import pallas as pl
from jax.experimental.pallas import tpu as pltpu
from jax.experimental.pallas import tpu_sc as plsc

_V = 1000000
_D = 64
_B = 4096
_T = 200
_SR = _V // 2
_AC = 384
_NA = _V // _AC
_AREM = _NA * _AC
_BC = 256
_TG = 8
_NB = (_T // _TG) * (_B // _BC)


def _mesh():
    return plsc.VectorSubcoreMesh(core_axis_name="c", subcore_axis_name="s")


@functools.cache
def _build():
    info = plsc.get_sparse_core_info()
    nc = info.num_cores
    nw = nc * info.num_subcores

    @functools.partial(
        pl.kernel,
        mesh=_mesh(),
        out_type=jax.ShapeDtypeStruct((_SR, 128), jnp.float32),
        scratch_types=[
            pltpu.VMEM((_D, _AC), jnp.float32),
            pltpu.VMEM((_D, _AC), jnp.float32),
            pltpu.VMEM((_AC // 2, 128), jnp.float32),
            pltpu.VMEM((_AC // 2, 128), jnp.float32),
            pltpu.VMEM((32, 128), jnp.float32),
            pltpu.SemaphoreType.DMA,
            pltpu.SemaphoreType.DMA,
            pltpu.SemaphoreType.DMA,
            pltpu.SemaphoreType.DMA,
        ],
        compiler_params=pltpu.CompilerParams(
            use_tc_tiling_on_sc=True, needs_layout_passes=False
        ),
    )
    def a_kernel(tt_hbm, tail_hbm, scr_hbm, stage0_v, stage1_v, rowsT0_v,
                 rowsT1_v, tail_v, a0sem, a1sem, o0sem, o1sem):
        wid = lax.axis_index("s") * nc + lax.axis_index("c")
        lio = lax.iota(jnp.int32, 16)
        perms = [(lio + k) & 15 for k in range(16)]
        n_mine = (_NA - wid + nw - 1) // nw
        n_pairs = n_mine // 2

        def in_copy(m, stage, sem):
            i0 = pl.multiple_of((wid + m * nw) * _AC, _AC)
            return pltpu.make_async_copy(tt_hbm.at[:, pl.ds(i0, _AC)], stage,
                                         sem)

        def out_copy(m, rowsT, sem):
            r0 = pl.multiple_of((wid + m * nw) * (_AC // 2), _AC // 2)
            return pltpu.make_async_copy(rowsT, scr_hbm.at[pl.ds(r0, _AC // 2), :],
                                         sem)

        def transpose(stage, rowsT):
            @plsc.parallel_loop(0, (_AC // 16) * 4, unroll=4)
            def block(q):
                jb = q // 4
                db = q % 4
                d_vec = db * 16 + lio
                jb16 = jnp.broadcast_to(jb * 16, (16,)).astype(jnp.int32)
                jb8 = jnp.broadcast_to(jb * 8, (16,)).astype(jnp.int32)
                for k in range(16):
                    j_vec = jb16 + perms[k]
                    v = plsc.load_gather(stage, [d_vec, j_vec])
                    row = jb8 + lax.shift_right_logical(perms[k], 1)
                    col = lax.shift_left(perms[k] & 1, 6) + d_vec
                    plsc.store_scatter(rowsT, [row, col], v)

        in_copy(0, stage0_v, a0sem).start()

        def pair(g, carry):
            m0 = 2 * g
            m1 = 2 * g + 1
            in_copy(m0, stage0_v, a0sem).wait()
            in_copy(m1, stage1_v, a1sem).start()

            @pl.when(g > 0)
            def _():
                out_copy(m0, rowsT0_v, o0sem).wait()

            transpose(stage0_v, rowsT0_v)
            out_copy(m0, rowsT0_v, o0sem).start()

            in_copy(m1, stage1_v, a1sem).wait()

            @pl.when(m1 + 1 < n_mine)
            def _():
                in_copy(m1 + 1, stage0_v, a0sem).start()

            @pl.when(g > 0)
            def _():
                out_copy(m1, rowsT1_v, o1sem).wait()

            transpose(stage1_v, rowsT1_v)
            out_copy(m1, rowsT1_v, o1sem).start()
            return carry

        lax.fori_loop(0, n_pairs, pair, 0)

        @pl.when(n_mine % 2 == 1)
        def _():
            mt = 2 * n_pairs
            in_copy(mt, stage0_v, a0sem).wait()

            @pl.when(n_pairs > 0)
            def _():
                out_copy(mt, rowsT0_v, o0sem).wait()

            transpose(stage0_v, rowsT0_v)
            out_copy(mt, rowsT0_v, o0sem).start()

        out_copy(0, rowsT0_v, o0sem).wait()

        @pl.when(n_pairs > 0)
        def _():
            out_copy(0, rowsT1_v, o1sem).wait()

        @pl.when(wid == nw - 1)
        def _():
            pltpu.sync_copy(tail_hbm, tail_v)
            pltpu.sync_copy(tail_v, scr_hbm.at[pl.ds(_AREM // 2, 32), :])

    @functools.partial(
        pl.kernel,
        mesh=_mesh(),
        out_type=jax.ShapeDtypeStruct((_T, _D, _B), jnp.float32),
        scratch_types=[
            pltpu.VMEM((_TG, _BC), jnp.int32),
            pltpu.VMEM((_TG, _BC), jnp.int32),
            pltpu.VMEM((_TG, _BC), jnp.int32),
            pltpu.VMEM((_BC, 128), jnp.float32),
            pltpu.VMEM((_BC, 128), jnp.float32),
            pltpu.VMEM((_D, _BC), jnp.float32),
            pltpu.VMEM((_D, _BC), jnp.float32),
            pltpu.SemaphoreType.DMA,
            pltpu.SemaphoreType.DMA,
            pltpu.SemaphoreType.DMA,
            pltpu.SemaphoreType.DMA,
        ],
        compiler_params=pltpu.CompilerParams(
            use_tc_tiling_on_sc=True, needs_layout_passes=False
        ),
    )
    def b_kernel(scr_hbm, ids_hbm, out_hbm, ids_v, idx2_v, par_v, rows0_v,
                 rows1_v, piece0_v, piece1_v, g0sem, g1sem, p0sem, p1sem):
        wid = lax.axis_index("s") * nc + lax.axis_index("c")
        lio = lax.iota(jnp.int32, 16)
        perms = [(lio + k) & 15 for k in range(16)]
        n_mine = (_NB - wid + nw - 1) // nw

        def unit(m, _):
            u = wid + m * nw
            g = u // (_B // _BC)
            bc = u % (_B // _BC)
            t0 = pl.multiple_of(g * _TG, _TG)
            b0 = pl.multiple_of(bc * _BC, _BC)
            pltpu.sync_copy(ids_hbm.at[pl.ds(t0, _TG), pl.ds(b0, _BC)], ids_v)

            @plsc.parallel_loop(0, _TG * (_BC // 16), unroll=4)
            def prep(q):
                r = q // (_BC // 16)
                c = q % (_BC // 16)
                v = ids_v[r, pl.ds(16 * c, 16)]
                idx2_v[r, pl.ds(16 * c, 16)] = lax.shift_right_logical(v, 1)
                par_v[r, pl.ds(16 * c, 16)] = lax.bitwise_and(v, 1)

            def g_fire(tl, rows, sem):
                for k in range(_BC // 128):
                    pltpu.async_copy(
                        scr_hbm.at[idx2_v.at[tl, pl.ds(128 * k, 128)]],
                        rows.at[pl.ds(128 * k, 128)],
                        sem,
                    )

            def g_wait(tl, rows, sem):
                for k in range(_BC // 128):
                    pltpu.make_async_copy(
                        scr_hbm.at[idx2_v.at[tl, pl.ds(128 * k, 128)]],
                        rows.at[pl.ds(128 * k, 128)],
                        sem,
                    ).wait()

            def out_slice(tl):
                return out_hbm.at[t0 + tl, :, pl.ds(b0, _BC)]

            g_fire(0, rows0_v, g0sem)
            for tl in range(_TG):
                rows, gs = (rows0_v, g0sem) if tl % 2 == 0 else (rows1_v, g1sem)
                piece, ps = ((piece0_v, p0sem) if tl % 2 == 0
                             else (piece1_v, p1sem))
                g_wait(tl, rows, gs)
                if tl + 1 < _TG:
                    nrows, ngs = ((rows1_v, g1sem) if tl % 2 == 0
                                  else (rows0_v, g0sem))
                    g_fire(tl + 1, nrows, ngs)
                if tl >= 2:
                    pltpu.make_async_copy(piece, out_slice(tl - 2), ps).wait()

                @plsc.parallel_loop(0, (_D // 16) * (_BC // 16), unroll=4)
                def bb_body(q):
                    db = q // (_BC // 16)
                    bb = q % (_BC // 16)
                    row = 16 * bb + lio
                    par64 = lax.shift_left(par_v[tl, pl.ds(16 * bb, 16)], 6)
                    for k in range(16):
                        dperm = db * 16 + perms[k]
                        v = plsc.load_gather(rows, [row, par64 + dperm])
                        plsc.store_scatter(piece, [dperm, row], v)

                pltpu.async_copy(piece, out_slice(tl), ps)
            pltpu.make_async_copy(piece0_v, out_slice(_TG - 2), p0sem).wait()
            pltpu.make_async_copy(piece1_v, out_slice(_TG - 1), p1sem).wait()
            return _

        lax.fori_loop(0, n_mine, unit, 0)

    return a_kernel, b_kernel


def kernel(input_ids, table):
    a_kernel, b_kernel = _build()
    tt = table.T
    tail = table[_AREM:].reshape(32, 128)
    scratch = a_kernel(tt, tail)
    out_t = b_kernel(scratch, input_ids.T)
    return jnp.transpose(out_t, (2, 0, 1))
